# Initial kernel scaffold; baseline (speedup 1.0000x reference)
import jax
import jax.numpy as jnp
from jax.experimental import pallas as pl


def kernel(z, pos, edge_index, batch, emb_table):
    raise NotImplementedError("write your pallas kernel here")


# speedup vs baseline: 1.9868x; 1.9868x over previous
import functools

import numpy as np
import jax
import jax.numpy as jnp
from jax import lax
from jax.experimental import pallas as pl
from jax.experimental.pallas import tpu as pltpu
from jax.experimental.pallas import tpu_sc as plsc

N_FEATURES = 128
Z_MAX = 100
R_CUT = 5.0
GAMMA = 10.0
N_NODES = 10000
N_EDGES = 320000

NC, NS, L = 2, 16, 16
NW = NC * NS

E_PER_W = N_EDGES // NW
Z_PAD = 10240
Z_PER_W = Z_PAD // NW

@functools.lru_cache(maxsize=1)
def _sc_kernels():
    mesh = plsc.VectorSubcoreMesh(
        core_axis_name="c", subcore_axis_name="s", num_cores=NC, num_subcores=NS
    )

    @functools.partial(
        pl.kernel,
        out_type=jax.ShapeDtypeStruct((N_EDGES,), jnp.float32),
        mesh=mesh,
        scratch_types=[
            pltpu.VMEM((N_NODES * 3,), jnp.float32),
            pltpu.VMEM((E_PER_W,), jnp.int32),
            pltpu.VMEM((E_PER_W,), jnp.int32),
            pltpu.VMEM((E_PER_W,), jnp.float32),
        ],
        compiler_params=pltpu.CompilerParams(needs_layout_passes=False),
    )
    def sc_dist(pos_hbm, src_hbm, dst_hbm, d2_hbm, pos_v, src_v, dst_v, d2_v):
        wid = lax.axis_index("s") * NC + lax.axis_index("c")
        base = wid * E_PER_W
        pltpu.sync_copy(pos_hbm, pos_v)
        pltpu.sync_copy(src_hbm.at[pl.ds(base, E_PER_W)], src_v)
        pltpu.sync_copy(dst_hbm.at[pl.ds(base, E_PER_W)], dst_v)

        three = jnp.full((L,), 3, jnp.int32)

        def body(i, carry):
            off = i * L
            s = src_v[pl.ds(off, L)] * three
            t = dst_v[pl.ds(off, L)] * three
            acc = jnp.zeros((L,), jnp.float32)
            for c in range(3):
                col = jnp.full((L,), c, jnp.int32)
                a = plsc.load_gather(pos_v, [s + col])
                b = plsc.load_gather(pos_v, [t + col])
                diff = a - b
                acc = acc + diff * diff
            d2_v[pl.ds(off, L)] = acc
            return carry

        lax.fori_loop(0, E_PER_W // L, body, 0)
        pltpu.sync_copy(d2_v, d2_hbm.at[pl.ds(base, E_PER_W)])

    @functools.partial(
        pl.kernel,
        out_type=jax.ShapeDtypeStruct((Z_PAD, N_FEATURES), jnp.float32),
        mesh=mesh,
        scratch_types=[
            pltpu.VMEM((Z_PER_W,), jnp.int32),
            pltpu.VMEM((Z_PER_W, N_FEATURES), jnp.float32),
            pltpu.SemaphoreType.DMA,
        ],
    )
    def sc_embed(z_hbm, table_hbm, out_hbm, idx_v, rows_v, sem):
        wid = lax.axis_index("s") * NC + lax.axis_index("c")
        base = wid * Z_PER_W
        pltpu.sync_copy(z_hbm.at[pl.ds(base, Z_PER_W)], idx_v)
        pltpu.async_copy(table_hbm.at[idx_v], rows_v, sem).wait()
        pltpu.sync_copy(rows_v, out_hbm.at[pl.ds(base, Z_PER_W)])

    return sc_dist, sc_embed


_RBF_ROWS = 3200
_N_BLOCKS = N_EDGES // _RBF_ROWS


def _rbf_body(d2_ref, out_ref):
    d2 = d2_ref[...]
    d = jnp.sqrt(d2 + 1e-12)
    cut = 0.5 * (jnp.cos(jnp.pi * d / R_CUT) + 1.0)
    cut = cut * (d < R_CUT).astype(jnp.float32)
    mu_i = lax.broadcasted_iota(jnp.int32, (1, N_FEATURES), 1)
    mu = mu_i.astype(jnp.float32) * jnp.float32(R_CUT / (N_FEATURES - 1))
    delta = d - mu
    out_ref[...] = cut * jnp.exp(-GAMMA * (delta * delta))


_rbf = pl.pallas_call(
    _rbf_body,
    grid=(_N_BLOCKS,),
    in_specs=[pl.BlockSpec((_RBF_ROWS, 1), lambda i: (i, 0))],
    out_specs=pl.BlockSpec((_RBF_ROWS, N_FEATURES), lambda i: (i, 0)),
    out_shape=jax.ShapeDtypeStruct((N_EDGES, N_FEATURES), jnp.float32),
)


def kernel(z, pos, edge_index, batch, emb_table):
    del batch
    z = z.astype(jnp.int32)
    edge_index = edge_index.astype(jnp.int32)
    pos = pos.astype(jnp.float32)
    emb_table = emb_table.astype(jnp.float32)

    sc_dist, sc_embed = _sc_kernels()
    d2 = sc_dist(pos.reshape(N_NODES * 3), edge_index[0], edge_index[1])
    dist_edge = _rbf(d2.reshape(N_EDGES, 1))

    z_pad = jnp.concatenate([z, jnp.zeros((Z_PAD - N_NODES,), jnp.int32)])
    atom_node = sc_embed(z_pad, emb_table)[:N_NODES]

    force_node = jnp.zeros((N_NODES, 3, N_FEATURES), jnp.float32)
    disp_node = jnp.zeros((N_NODES, 3, N_FEATURES), jnp.float32)
    return (atom_node, force_node, disp_node, dist_edge)


# speedup vs baseline: 4.8953x; 2.4640x over previous
import functools

import numpy as np
import jax
import jax.numpy as jnp
from jax import lax
from jax.experimental import pallas as pl
from jax.experimental.pallas import tpu as pltpu
from jax.experimental.pallas import tpu_sc as plsc

N_FEATURES = 128
Z_MAX = 100
R_CUT = 5.0
GAMMA = 10.0
N_NODES = 10000
N_EDGES = 320000

NC, NS, L = 2, 16, 16
NW = NC * NS

E_PER_W = N_EDGES // NW
Z_PAD = 10240
Z_PER_W = Z_PAD // NW

@functools.lru_cache(maxsize=1)
def _sc_kernels():
    mesh = plsc.VectorSubcoreMesh(
        core_axis_name="c", subcore_axis_name="s", num_cores=NC, num_subcores=NS
    )

    @functools.partial(
        pl.kernel,
        out_type=jax.ShapeDtypeStruct((N_EDGES,), jnp.float32),
        mesh=mesh,
        scratch_types=[
            pltpu.VMEM((N_NODES * 3,), jnp.float32),
            pltpu.VMEM((E_PER_W,), jnp.int32),
            pltpu.VMEM((E_PER_W,), jnp.int32),
            pltpu.VMEM((E_PER_W,), jnp.float32),
        ],
        compiler_params=pltpu.CompilerParams(needs_layout_passes=False),
    )
    def sc_dist(pos_hbm, src_hbm, dst_hbm, d2_hbm, pos_v, src_v, dst_v, d2_v):
        wid = lax.axis_index("s") * NC + lax.axis_index("c")
        base = wid * E_PER_W
        pltpu.sync_copy(pos_hbm, pos_v)
        pltpu.sync_copy(src_hbm.at[pl.ds(base, E_PER_W)], src_v)
        pltpu.sync_copy(dst_hbm.at[pl.ds(base, E_PER_W)], dst_v)

        three = jnp.full((L,), 3, jnp.int32)

        def body(i, carry):
            off = i * L
            s = src_v[pl.ds(off, L)] * three
            t = dst_v[pl.ds(off, L)] * three
            acc = jnp.zeros((L,), jnp.float32)
            for c in range(3):
                col = jnp.full((L,), c, jnp.int32)
                a = plsc.load_gather(pos_v, [s + col])
                b = plsc.load_gather(pos_v, [t + col])
                diff = a - b
                acc = acc + diff * diff
            d2_v[pl.ds(off, L)] = acc
            return carry

        lax.fori_loop(0, E_PER_W // L, body, 0)
        pltpu.sync_copy(d2_v, d2_hbm.at[pl.ds(base, E_PER_W)])

    @functools.partial(
        pl.kernel,
        out_type=jax.ShapeDtypeStruct((Z_PAD, N_FEATURES), jnp.float32),
        mesh=mesh,
        scratch_types=[
            pltpu.VMEM((Z_PER_W,), jnp.int32),
            pltpu.VMEM((Z_PER_W, N_FEATURES), jnp.float32),
            pltpu.SemaphoreType.DMA,
        ],
    )
    def sc_embed(z_hbm, table_hbm, out_hbm, idx_v, rows_v, sem):
        wid = lax.axis_index("s") * NC + lax.axis_index("c")
        base = wid * Z_PER_W
        pltpu.sync_copy(z_hbm.at[pl.ds(base, Z_PER_W)], idx_v)
        pltpu.async_copy(table_hbm.at[idx_v], rows_v, sem).wait()
        pltpu.sync_copy(rows_v, out_hbm.at[pl.ds(base, Z_PER_W)])

    return sc_dist, sc_embed


_RBF_ROWS = 3200
_N_BLOCKS = N_EDGES // _RBF_ROWS
_PRE_ROWS = N_EDGES // N_FEATURES


def _mu_row():
    mu_i = lax.broadcasted_iota(jnp.int32, (1, N_FEATURES), 1)
    return mu_i.astype(jnp.float32) * jnp.float32(R_CUT / (N_FEATURES - 1))


def _pre_body(d2_ref, b_ref, h_ref):
    d2 = d2_ref[...]
    d = jnp.sqrt(d2 + 1e-12)
    cut = 0.5 * (jnp.cos(jnp.pi * d / R_CUT) + 1.0)
    cut = jnp.where(d < R_CUT, cut, 0.0)
    cut = jnp.maximum(cut, 1e-37)
    b_ref[...] = (2.0 * GAMMA) * d
    h_ref[...] = jnp.log(cut) - GAMMA * (d * d)


_pre = pl.pallas_call(
    _pre_body,
    out_shape=(
        jax.ShapeDtypeStruct((_PRE_ROWS, N_FEATURES), jnp.float32),
        jax.ShapeDtypeStruct((_PRE_ROWS, N_FEATURES), jnp.float32),
    ),
)


def _rbf_body(a_ref, out_ref):
    A = a_ref[...]
    mu = _mu_row()
    ones = jnp.ones((1, N_FEATURES), jnp.float32)
    B = jnp.concatenate([mu, ones], axis=0)
    acc = lax.dot_general(
        A, B, (((1,), (0,)), ((), ())),
        precision=lax.Precision.HIGHEST,
        preferred_element_type=jnp.float32,
    )
    g = GAMMA * (mu * mu)
    out_ref[...] = jnp.exp(acc - g)


_rbf = pl.pallas_call(
    _rbf_body,
    grid=(_N_BLOCKS,),
    in_specs=[pl.BlockSpec((_RBF_ROWS, 2), lambda i: (i, 0))],
    out_specs=pl.BlockSpec((_RBF_ROWS, N_FEATURES), lambda i: (i, 0)),
    out_shape=jax.ShapeDtypeStruct((N_EDGES, N_FEATURES), jnp.float32),
)


def kernel(z, pos, edge_index, batch, emb_table):
    del batch
    z = z.astype(jnp.int32)
    edge_index = edge_index.astype(jnp.int32)
    pos = pos.astype(jnp.float32)
    emb_table = emb_table.astype(jnp.float32)

    sc_dist, sc_embed = _sc_kernels()
    d2 = sc_dist(pos.reshape(N_NODES * 3), edge_index[0], edge_index[1])
    b, h = _pre(d2.reshape(_PRE_ROWS, N_FEATURES))
    A = jnp.stack([b.reshape(N_EDGES), h.reshape(N_EDGES)], axis=-1)
    dist_edge = _rbf(A)

    z_pad = jnp.concatenate([z, jnp.zeros((Z_PAD - N_NODES,), jnp.int32)])
    atom_node = sc_embed(z_pad, emb_table)[:N_NODES]

    force_node = jnp.zeros((N_NODES, 3, N_FEATURES), jnp.float32)
    disp_node = jnp.zeros((N_NODES, 3, N_FEATURES), jnp.float32)
    return (atom_node, force_node, disp_node, dist_edge)


# speedup vs baseline: 4.9253x; 1.0061x over previous
import functools

import numpy as np
import jax
import jax.numpy as jnp
from jax import lax
from jax.experimental import pallas as pl
from jax.experimental.pallas import tpu as pltpu
from jax.experimental.pallas import tpu_sc as plsc

N_FEATURES = 128
Z_MAX = 100
R_CUT = 5.0
GAMMA = 10.0
N_NODES = 10000
N_EDGES = 320000

NC, NS, L = 2, 16, 16
NW = NC * NS

E_PER_W = N_EDGES // NW
Z_PAD = 10240
Z_PER_W = Z_PAD // NW

@functools.lru_cache(maxsize=1)
def _sc_kernels():
    mesh = plsc.VectorSubcoreMesh(
        core_axis_name="c", subcore_axis_name="s", num_cores=NC, num_subcores=NS
    )

    @functools.partial(
        pl.kernel,
        out_type=jax.ShapeDtypeStruct((N_EDGES,), jnp.float32),
        mesh=mesh,
        scratch_types=[
            pltpu.VMEM((N_NODES * 3,), jnp.float32),
            pltpu.VMEM((E_PER_W,), jnp.int32),
            pltpu.VMEM((E_PER_W,), jnp.int32),
            pltpu.VMEM((E_PER_W,), jnp.float32),
        ],
        compiler_params=pltpu.CompilerParams(needs_layout_passes=False),
    )
    def sc_dist(pos_hbm, src_hbm, dst_hbm, d2_hbm, pos_v, src_v, dst_v, d2_v):
        wid = lax.axis_index("s") * NC + lax.axis_index("c")
        base = wid * E_PER_W
        pltpu.sync_copy(pos_hbm, pos_v)
        pltpu.sync_copy(src_hbm.at[pl.ds(base, E_PER_W)], src_v)
        pltpu.sync_copy(dst_hbm.at[pl.ds(base, E_PER_W)], dst_v)

        three = jnp.full((L,), 3, jnp.int32)

        @plsc.parallel_loop(0, E_PER_W // L, 1, unroll=8)
        def body(i):
            off = i * L
            s = src_v[pl.ds(off, L)] * three
            t = dst_v[pl.ds(off, L)] * three
            acc = jnp.zeros((L,), jnp.float32)
            for c in range(3):
                col = jnp.full((L,), c, jnp.int32)
                a = plsc.load_gather(pos_v, [s + col])
                b = plsc.load_gather(pos_v, [t + col])
                diff = a - b
                acc = acc + diff * diff
            d2_v[pl.ds(off, L)] = acc
        pltpu.sync_copy(d2_v, d2_hbm.at[pl.ds(base, E_PER_W)])

    @functools.partial(
        pl.kernel,
        out_type=jax.ShapeDtypeStruct((Z_PAD, N_FEATURES), jnp.float32),
        mesh=mesh,
        scratch_types=[
            pltpu.VMEM((Z_PER_W,), jnp.int32),
            pltpu.VMEM((Z_PER_W, N_FEATURES), jnp.float32),
            pltpu.SemaphoreType.DMA,
        ],
    )
    def sc_embed(z_hbm, table_hbm, out_hbm, idx_v, rows_v, sem):
        wid = lax.axis_index("s") * NC + lax.axis_index("c")
        base = wid * Z_PER_W
        pltpu.sync_copy(z_hbm.at[pl.ds(base, Z_PER_W)], idx_v)
        pltpu.async_copy(table_hbm.at[idx_v], rows_v, sem).wait()
        pltpu.sync_copy(rows_v, out_hbm.at[pl.ds(base, Z_PER_W)])

    return sc_dist, sc_embed


_RBF_ROWS = 3200
_N_BLOCKS = N_EDGES // _RBF_ROWS
_PRE_ROWS = N_EDGES // N_FEATURES


def _mu_row():
    mu_i = lax.broadcasted_iota(jnp.int32, (1, N_FEATURES), 1)
    return mu_i.astype(jnp.float32) * jnp.float32(R_CUT / (N_FEATURES - 1))


def _pre_body(d2_ref, b_ref, h_ref):
    d2 = d2_ref[...]
    d = jnp.sqrt(d2 + 1e-12)
    cut = 0.5 * (jnp.cos(jnp.pi * d / R_CUT) + 1.0)
    cut = jnp.where(d < R_CUT, cut, 0.0)
    cut = jnp.maximum(cut, 1e-37)
    b_ref[...] = (2.0 * GAMMA) * d
    h_ref[...] = jnp.log(cut) - GAMMA * (d * d)


_pre = pl.pallas_call(
    _pre_body,
    out_shape=(
        jax.ShapeDtypeStruct((_PRE_ROWS, N_FEATURES), jnp.float32),
        jax.ShapeDtypeStruct((_PRE_ROWS, N_FEATURES), jnp.float32),
    ),
)


def _rbf_body(a_ref, out_ref):
    A = a_ref[...]
    mu = _mu_row()
    ones = jnp.ones((1, N_FEATURES), jnp.float32)
    B = jnp.concatenate([mu, ones], axis=0)
    acc = lax.dot_general(
        A, B, (((1,), (0,)), ((), ())),
        precision=lax.Precision.HIGHEST,
        preferred_element_type=jnp.float32,
    )
    g = GAMMA * (mu * mu)
    out_ref[...] = jnp.exp(acc - g)


_rbf = pl.pallas_call(
    _rbf_body,
    grid=(_N_BLOCKS,),
    in_specs=[pl.BlockSpec((_RBF_ROWS, 2), lambda i: (i, 0))],
    out_specs=pl.BlockSpec((_RBF_ROWS, N_FEATURES), lambda i: (i, 0)),
    out_shape=jax.ShapeDtypeStruct((N_EDGES, N_FEATURES), jnp.float32),
)


def kernel(z, pos, edge_index, batch, emb_table):
    del batch
    z = z.astype(jnp.int32)
    edge_index = edge_index.astype(jnp.int32)
    pos = pos.astype(jnp.float32)
    emb_table = emb_table.astype(jnp.float32)

    sc_dist, sc_embed = _sc_kernels()
    d2 = sc_dist(pos.reshape(N_NODES * 3), edge_index[0], edge_index[1])
    b, h = _pre(d2.reshape(_PRE_ROWS, N_FEATURES))
    A = jnp.stack([b.reshape(N_EDGES), h.reshape(N_EDGES)], axis=-1)
    dist_edge = _rbf(A)

    z_pad = jnp.concatenate([z, jnp.zeros((Z_PAD - N_NODES,), jnp.int32)])
    atom_node = sc_embed(z_pad, emb_table)[:N_NODES]

    force_node = jnp.zeros((N_NODES, 3, N_FEATURES), jnp.float32)
    disp_node = jnp.zeros((N_NODES, 3, N_FEATURES), jnp.float32)
    return (atom_node, force_node, disp_node, dist_edge)


# speedup vs baseline: 6.9993x; 1.4211x over previous
import functools

import jax
import jax.numpy as jnp
from jax import lax
from jax.experimental import pallas as pl
from jax.experimental.pallas import tpu as pltpu
from jax.experimental.pallas import tpu_sc as plsc

N_FEATURES = 128
Z_MAX = 100
R_CUT = 5.0
GAMMA = 10.0
N_NODES = 10000
N_EDGES = 320000

NC, NS, L = 2, 16, 16
NW = NC * NS

E_PER_W = N_EDGES // NW
Z_PAD = 10240
Z_PER_W = Z_PAD // NW


@functools.lru_cache(maxsize=1)
def _sc_kernel():
    mesh = plsc.VectorSubcoreMesh(
        core_axis_name="c", subcore_axis_name="s", num_cores=NC, num_subcores=NS
    )

    @functools.partial(
        pl.kernel,
        out_type=(
            jax.ShapeDtypeStruct((N_EDGES,), jnp.float32),
            jax.ShapeDtypeStruct((Z_PAD, N_FEATURES), jnp.float32),
        ),
        mesh=mesh,
        scratch_types=[
            pltpu.VMEM((N_NODES * 3,), jnp.float32),
            pltpu.VMEM((E_PER_W,), jnp.int32),
            pltpu.VMEM((E_PER_W,), jnp.int32),
            pltpu.VMEM((E_PER_W,), jnp.float32),
            pltpu.VMEM((Z_PER_W,), jnp.int32),
            pltpu.VMEM((Z_PER_W, N_FEATURES), jnp.float32),
            pltpu.SemaphoreType.DMA,
        ],
        compiler_params=pltpu.CompilerParams(needs_layout_passes=False),
    )
    def sc_fused(pos_hbm, eidx_hbm, z_hbm, table_hbm, d2_hbm, emb_hbm,
                 pos_v, src_v, dst_v, d2_v, idx_v, rows_v, sem):
        wid = lax.axis_index("s") * NC + lax.axis_index("c")
        zbase = wid * Z_PER_W
        pltpu.sync_copy(z_hbm.at[pl.ds(zbase, Z_PER_W)], idx_v)
        emb_cp = pltpu.async_copy(table_hbm.at[idx_v], rows_v, sem)

        base = wid * E_PER_W
        pltpu.sync_copy(pos_hbm, pos_v)
        pltpu.sync_copy(eidx_hbm.at[pl.ds(base, E_PER_W)], src_v)
        pltpu.sync_copy(eidx_hbm.at[pl.ds(N_EDGES + base, E_PER_W)], dst_v)

        three = jnp.full((L,), 3, jnp.int32)

        @plsc.parallel_loop(0, E_PER_W // L, 1, unroll=8)
        def body(i):
            off = i * L
            s = src_v[pl.ds(off, L)] * three
            t = dst_v[pl.ds(off, L)] * three
            acc = jnp.zeros((L,), jnp.float32)
            for c in range(3):
                col = jnp.full((L,), c, jnp.int32)
                a = plsc.load_gather(pos_v, [s + col])
                b = plsc.load_gather(pos_v, [t + col])
                diff = a - b
                acc = acc + diff * diff
            d2_v[pl.ds(off, L)] = acc

        pltpu.sync_copy(d2_v, d2_hbm.at[pl.ds(base, E_PER_W)])
        emb_cp.wait()
        pltpu.sync_copy(rows_v, emb_hbm.at[pl.ds(zbase, Z_PER_W)])

    return sc_fused


_RBF_ROWS = 3200
_N_BLOCKS = N_EDGES // _RBF_ROWS
_PRE_ROWS = N_EDGES // N_FEATURES


def _mu_row():
    mu_i = lax.broadcasted_iota(jnp.int32, (1, N_FEATURES), 1)
    return mu_i.astype(jnp.float32) * jnp.float32(R_CUT / (N_FEATURES - 1))


def _pre_body(d2_ref, a_ref):
    d2 = d2_ref[...]
    d = jnp.sqrt(d2 + 1e-12)
    cut = 0.5 * (jnp.cos(jnp.pi * d / R_CUT) + 1.0)
    cut = jnp.where(d < R_CUT, cut, 0.0)
    cut = jnp.maximum(cut, 1e-37)
    a_ref[0:_PRE_ROWS, :] = (2.0 * GAMMA) * d
    a_ref[_PRE_ROWS:2 * _PRE_ROWS, :] = jnp.log(cut) - GAMMA * (d * d)


_pre = pl.pallas_call(
    _pre_body,
    out_shape=jax.ShapeDtypeStruct((2 * _PRE_ROWS, N_FEATURES), jnp.float32),
)


def _rbf_body(a_ref, out_ref):
    A = a_ref[...]
    mu = _mu_row()
    ones = jnp.ones((1, N_FEATURES), jnp.float32)
    B = jnp.concatenate([mu, ones], axis=0)
    acc = lax.dot_general(
        A, B, (((0,), (0,)), ((), ())),
        precision=lax.Precision.HIGHEST,
        preferred_element_type=jnp.float32,
    )
    g = GAMMA * (mu * mu)
    out_ref[...] = jnp.exp(acc - g)


_rbf = pl.pallas_call(
    _rbf_body,
    grid=(_N_BLOCKS,),
    in_specs=[pl.BlockSpec((2, _RBF_ROWS), lambda i: (0, i))],
    out_specs=pl.BlockSpec((_RBF_ROWS, N_FEATURES), lambda i: (i, 0)),
    out_shape=jax.ShapeDtypeStruct((N_EDGES, N_FEATURES), jnp.float32),
)


def kernel(z, pos, edge_index, batch, emb_table):
    del batch
    z = z.astype(jnp.int32)
    edge_index = edge_index.astype(jnp.int32)
    pos = pos.astype(jnp.float32)
    emb_table = emb_table.astype(jnp.float32)

    z_pad = jnp.concatenate([z, jnp.zeros((Z_PAD - N_NODES,), jnp.int32)])
    d2, emb = _sc_kernel()(pos.reshape(N_NODES * 3),
                           edge_index.reshape(2 * N_EDGES), z_pad, emb_table)
    A = _pre(d2.reshape(_PRE_ROWS, N_FEATURES)).reshape(2, N_EDGES)
    dist_edge = _rbf(A)
    atom_node = emb[:N_NODES]

    force_node = jnp.zeros((N_NODES, 3, N_FEATURES), jnp.float32)
    disp_node = jnp.zeros((N_NODES, 3, N_FEATURES), jnp.float32)
    return (atom_node, force_node, disp_node, dist_edge)


# speedup vs baseline: 7.3842x; 1.0550x over previous
import functools

import jax
import jax.numpy as jnp
from jax import lax
from jax.experimental import pallas as pl
from jax.experimental.pallas import tpu as pltpu
from jax.experimental.pallas import tpu_sc as plsc

N_FEATURES = 128
Z_MAX = 100
R_CUT = 5.0
GAMMA = 10.0
N_NODES = 10000
N_EDGES = 320000

NC, NS, L = 2, 16, 16
NW = NC * NS

E_PER_W = N_EDGES // NW
Z_PAD = 10240
Z_PER_W = Z_PAD // NW


@functools.lru_cache(maxsize=1)
def _sc_kernel():
    mesh = plsc.VectorSubcoreMesh(
        core_axis_name="c", subcore_axis_name="s", num_cores=NC, num_subcores=NS
    )

    @functools.partial(
        pl.kernel,
        out_type=(
            jax.ShapeDtypeStruct((N_EDGES,), jnp.float32),
            jax.ShapeDtypeStruct((Z_PAD, N_FEATURES), jnp.float32),
        ),
        mesh=mesh,
        scratch_types=[
            pltpu.VMEM((N_NODES * 3,), jnp.float32),
            pltpu.VMEM((E_PER_W,), jnp.int32),
            pltpu.VMEM((E_PER_W,), jnp.int32),
            pltpu.VMEM((E_PER_W,), jnp.float32),
            pltpu.VMEM((Z_PER_W,), jnp.int32),
            pltpu.VMEM((Z_PER_W, N_FEATURES), jnp.float32),
            pltpu.SemaphoreType.DMA,
        ],
        compiler_params=pltpu.CompilerParams(needs_layout_passes=False),
    )
    def sc_fused(pos_hbm, eidx_hbm, z_hbm, table_hbm, d2_hbm, emb_hbm,
                 pos_v, src_v, dst_v, d2_v, idx_v, rows_v, sem):
        wid = lax.axis_index("s") * NC + lax.axis_index("c")
        zbase = wid * Z_PER_W
        pltpu.sync_copy(z_hbm.at[pl.ds(zbase, Z_PER_W)], idx_v)
        emb_cp = pltpu.async_copy(table_hbm.at[idx_v], rows_v, sem)

        base = wid * E_PER_W
        pltpu.sync_copy(pos_hbm, pos_v)
        pltpu.sync_copy(eidx_hbm.at[pl.ds(base, E_PER_W)], src_v)
        pltpu.sync_copy(eidx_hbm.at[pl.ds(N_EDGES + base, E_PER_W)], dst_v)

        three = jnp.full((L,), 3, jnp.int32)

        @plsc.parallel_loop(0, E_PER_W // L, 1, unroll=8)
        def body(i):
            off = i * L
            s = src_v[pl.ds(off, L)] * three
            t = dst_v[pl.ds(off, L)] * three
            acc = jnp.zeros((L,), jnp.float32)
            for c in range(3):
                col = jnp.full((L,), c, jnp.int32)
                a = plsc.load_gather(pos_v, [s + col])
                b = plsc.load_gather(pos_v, [t + col])
                diff = a - b
                acc = acc + diff * diff
            d2_v[pl.ds(off, L)] = acc

        pltpu.sync_copy(d2_v, d2_hbm.at[pl.ds(base, E_PER_W)])
        emb_cp.wait()
        pltpu.sync_copy(rows_v, emb_hbm.at[pl.ds(zbase, Z_PER_W)])

    return sc_fused


_RBF_ROWS = 6400
_N_BLOCKS = N_EDGES // _RBF_ROWS
_PRE_ROWS = N_EDGES // N_FEATURES


def _mu_row():
    mu_i = lax.broadcasted_iota(jnp.int32, (1, N_FEATURES), 1)
    return mu_i.astype(jnp.float32) * jnp.float32(R_CUT / (N_FEATURES - 1))


def _pre_body(d2_ref, a_ref):
    d2 = d2_ref[...]
    d = jnp.sqrt(d2 + 1e-12)
    cut = 0.5 * (jnp.cos(jnp.pi * d / R_CUT) + 1.0)
    cut = jnp.where(d < R_CUT, cut, 0.0)
    cut = jnp.maximum(cut, 1e-37)
    a_ref[0:_PRE_ROWS, :] = (2.0 * GAMMA) * d
    a_ref[_PRE_ROWS:2 * _PRE_ROWS, :] = jnp.log(cut) - GAMMA * (d * d)


_pre = pl.pallas_call(
    _pre_body,
    out_shape=jax.ShapeDtypeStruct((2 * _PRE_ROWS, N_FEATURES), jnp.float32),
)


def _rbf_body(a_ref, out_ref):
    A = a_ref[...]
    mu = _mu_row()
    ones = jnp.ones((1, N_FEATURES), jnp.float32)
    B = jnp.concatenate([mu, ones], axis=0)
    acc = lax.dot_general(
        A, B, (((0,), (0,)), ((), ())),
        precision=lax.Precision.HIGHEST,
        preferred_element_type=jnp.float32,
    )
    g = GAMMA * (mu * mu)
    out_ref[...] = jnp.exp(acc - g)


_rbf = pl.pallas_call(
    _rbf_body,
    grid=(_N_BLOCKS,),
    in_specs=[pl.BlockSpec((2, _RBF_ROWS), lambda i: (0, i))],
    out_specs=pl.BlockSpec((_RBF_ROWS, N_FEATURES), lambda i: (i, 0)),
    out_shape=jax.ShapeDtypeStruct((N_EDGES, N_FEATURES), jnp.float32),
)


def kernel(z, pos, edge_index, batch, emb_table):
    del batch
    z = z.astype(jnp.int32)
    edge_index = edge_index.astype(jnp.int32)
    pos = pos.astype(jnp.float32)
    emb_table = emb_table.astype(jnp.float32)

    z_pad = jnp.concatenate([z, jnp.zeros((Z_PAD - N_NODES,), jnp.int32)])
    d2, emb = _sc_kernel()(pos.reshape(N_NODES * 3),
                           edge_index.reshape(2 * N_EDGES), z_pad, emb_table)
    A = _pre(d2.reshape(_PRE_ROWS, N_FEATURES)).reshape(2, N_EDGES)
    dist_edge = _rbf(A)
    atom_node = emb[:N_NODES]

    force_node = jnp.zeros((N_NODES, 3, N_FEATURES), jnp.float32)
    disp_node = jnp.zeros((N_NODES, 3, N_FEATURES), jnp.float32)
    return (atom_node, force_node, disp_node, dist_edge)


# speedup vs baseline: 7.5443x; 1.0217x over previous
import functools

import jax
import jax.numpy as jnp
from jax import lax
from jax.experimental import pallas as pl
from jax.experimental.pallas import tpu as pltpu
from jax.experimental.pallas import tpu_sc as plsc

N_FEATURES = 128
Z_MAX = 100
R_CUT = 5.0
GAMMA = 10.0
N_NODES = 10000
N_EDGES = 320000

NC, NS, L = 2, 16, 16
NW = NC * NS

E_PER_W = N_EDGES // NW
Z_PAD = 10240
Z_PER_W = Z_PAD // NW


@functools.lru_cache(maxsize=1)
def _sc_kernel():
    mesh = plsc.VectorSubcoreMesh(
        core_axis_name="c", subcore_axis_name="s", num_cores=NC, num_subcores=NS
    )

    @functools.partial(
        pl.kernel,
        out_type=(
            jax.ShapeDtypeStruct((N_EDGES,), jnp.float32),
            jax.ShapeDtypeStruct((Z_PAD, N_FEATURES), jnp.float32),
        ),
        mesh=mesh,
        scratch_types=[
            pltpu.VMEM((N_NODES * 3,), jnp.float32),
            pltpu.VMEM((E_PER_W,), jnp.int32),
            pltpu.VMEM((E_PER_W,), jnp.int32),
            pltpu.VMEM((E_PER_W,), jnp.float32),
            pltpu.VMEM((Z_PER_W,), jnp.int32),
            pltpu.VMEM((Z_PER_W, N_FEATURES), jnp.float32),
            pltpu.SemaphoreType.DMA,
        ],
        compiler_params=pltpu.CompilerParams(needs_layout_passes=False),
    )
    def sc_fused(pos_hbm, eidx_hbm, z_hbm, table_hbm, d2_hbm, emb_hbm,
                 pos_v, src_v, dst_v, d2_v, idx_v, rows_v, sem):
        wid = lax.axis_index("s") * NC + lax.axis_index("c")
        zbase = wid * Z_PER_W
        pltpu.sync_copy(z_hbm.at[pl.ds(zbase, Z_PER_W)], idx_v)
        emb_cp = pltpu.async_copy(table_hbm.at[idx_v], rows_v, sem)

        base = wid * E_PER_W
        pltpu.sync_copy(pos_hbm, pos_v)
        pltpu.sync_copy(eidx_hbm.at[pl.ds(base, E_PER_W)], src_v)
        pltpu.sync_copy(eidx_hbm.at[pl.ds(N_EDGES + base, E_PER_W)], dst_v)

        three = jnp.full((L,), 3, jnp.int32)

        @plsc.parallel_loop(0, E_PER_W // L, 1, unroll=8)
        def body(i):
            off = i * L
            s = src_v[pl.ds(off, L)] * three
            t = dst_v[pl.ds(off, L)] * three
            acc = jnp.zeros((L,), jnp.float32)
            for c in range(3):
                col = jnp.full((L,), c, jnp.int32)
                a = plsc.load_gather(pos_v, [s + col])
                b = plsc.load_gather(pos_v, [t + col])
                diff = a - b
                acc = acc + diff * diff
            d2_v[pl.ds(off, L)] = acc

        pltpu.sync_copy(d2_v, d2_hbm.at[pl.ds(base, E_PER_W)])
        emb_cp.wait()
        pltpu.sync_copy(rows_v, emb_hbm.at[pl.ds(zbase, Z_PER_W)])

    return sc_fused


_RBF_ROWS = 12800
_N_BLOCKS = N_EDGES // _RBF_ROWS
_PRE_ROWS = N_EDGES // N_FEATURES


def _mu_row():
    mu_i = lax.broadcasted_iota(jnp.int32, (1, N_FEATURES), 1)
    return mu_i.astype(jnp.float32) * jnp.float32(R_CUT / (N_FEATURES - 1))


def _pre_body(d2_ref, a_ref):
    d2 = d2_ref[...]
    d = jnp.sqrt(d2 + 1e-12)
    cut = 0.5 * (jnp.cos(jnp.pi * d / R_CUT) + 1.0)
    cut = jnp.where(d < R_CUT, cut, 0.0)
    cut = jnp.maximum(cut, 1e-37)
    a_ref[0:_PRE_ROWS, :] = (2.0 * GAMMA) * d
    a_ref[_PRE_ROWS:2 * _PRE_ROWS, :] = jnp.log(cut) - GAMMA * (d * d)


_pre = pl.pallas_call(
    _pre_body,
    out_shape=jax.ShapeDtypeStruct((2 * _PRE_ROWS, N_FEATURES), jnp.float32),
)


def _rbf_body(a_ref, out_ref):
    A = a_ref[...]
    mu = _mu_row()
    ones = jnp.ones((1, N_FEATURES), jnp.float32)
    B = jnp.concatenate([mu, ones], axis=0)
    acc = lax.dot_general(
        A, B, (((0,), (0,)), ((), ())),
        precision=lax.Precision.HIGHEST,
        preferred_element_type=jnp.float32,
    )
    g = GAMMA * (mu * mu)
    out_ref[...] = jnp.exp(acc - g)


_rbf = pl.pallas_call(
    _rbf_body,
    grid=(_N_BLOCKS,),
    in_specs=[pl.BlockSpec((2, _RBF_ROWS), lambda i: (0, i))],
    out_specs=pl.BlockSpec((_RBF_ROWS, N_FEATURES), lambda i: (i, 0)),
    out_shape=jax.ShapeDtypeStruct((N_EDGES, N_FEATURES), jnp.float32),
)


def kernel(z, pos, edge_index, batch, emb_table):
    del batch
    z = z.astype(jnp.int32)
    edge_index = edge_index.astype(jnp.int32)
    pos = pos.astype(jnp.float32)
    emb_table = emb_table.astype(jnp.float32)

    z_pad = jnp.concatenate([z, jnp.zeros((Z_PAD - N_NODES,), jnp.int32)])
    d2, emb = _sc_kernel()(pos.reshape(N_NODES * 3),
                           edge_index.reshape(2 * N_EDGES), z_pad, emb_table)
    A = _pre(d2.reshape(_PRE_ROWS, N_FEATURES)).reshape(2, N_EDGES)
    dist_edge = _rbf(A)
    atom_node = emb[:N_NODES]

    force_node = jnp.zeros((N_NODES, 3, N_FEATURES), jnp.float32)
    disp_node = jnp.zeros((N_NODES, 3, N_FEATURES), jnp.float32)
    return (atom_node, force_node, disp_node, dist_edge)


# speedup vs baseline: 11.7461x; 1.5569x over previous
import functools

import jax
import jax.numpy as jnp
from jax import lax
from jax.experimental import pallas as pl
from jax.experimental.pallas import tpu as pltpu
from jax.experimental.pallas import tpu_sc as plsc

N_FEATURES = 128
Z_MAX = 100
R_CUT = 5.0
GAMMA = 10.0
N_NODES = 10000
N_EDGES = 320000

NC, NS, L = 2, 16, 16
NW = NC * NS

E_PER_W = N_EDGES // NW
Z_PAD = 10240
Z_PER_W = Z_PAD // NW


@functools.lru_cache(maxsize=1)
def _sc_kernel():
    mesh = plsc.VectorSubcoreMesh(
        core_axis_name="c", subcore_axis_name="s", num_cores=NC, num_subcores=NS
    )

    @functools.partial(
        pl.kernel,
        out_type=(
            jax.ShapeDtypeStruct((N_EDGES,), jnp.float32),
            jax.ShapeDtypeStruct((Z_PAD, N_FEATURES), jnp.float32),
        ),
        mesh=mesh,
        scratch_types=[
            pltpu.VMEM((N_NODES * 3,), jnp.float32),
            pltpu.VMEM((E_PER_W,), jnp.int32),
            pltpu.VMEM((E_PER_W,), jnp.int32),
            pltpu.VMEM((E_PER_W,), jnp.float32),
            pltpu.VMEM((Z_PER_W,), jnp.int32),
            pltpu.VMEM((Z_PER_W, N_FEATURES), jnp.float32),
            pltpu.SemaphoreType.DMA,
        ],
        compiler_params=pltpu.CompilerParams(needs_layout_passes=False),
    )
    def sc_fused(pos_hbm, eidx_hbm, z_hbm, table_hbm, d2_hbm, emb_hbm,
                 pos_v, src_v, dst_v, d2_v, idx_v, rows_v, sem):
        wid = lax.axis_index("s") * NC + lax.axis_index("c")
        zbase = wid * Z_PER_W
        pltpu.sync_copy(z_hbm.at[pl.ds(zbase, Z_PER_W)], idx_v)
        emb_cp = pltpu.async_copy(table_hbm.at[idx_v], rows_v, sem)

        base = wid * E_PER_W
        pltpu.sync_copy(pos_hbm, pos_v)
        pltpu.sync_copy(eidx_hbm.at[pl.ds(base, E_PER_W)], src_v)
        pltpu.sync_copy(eidx_hbm.at[pl.ds(N_EDGES + base, E_PER_W)], dst_v)

        three = jnp.full((L,), 3, jnp.int32)

        @plsc.parallel_loop(0, E_PER_W // L, 1, unroll=8)
        def body(i):
            off = i * L
            s = src_v[pl.ds(off, L)] * three
            t = dst_v[pl.ds(off, L)] * three
            acc = jnp.zeros((L,), jnp.float32)
            for c in range(3):
                col = jnp.full((L,), c, jnp.int32)
                a = plsc.load_gather(pos_v, [s + col])
                b = plsc.load_gather(pos_v, [t + col])
                diff = a - b
                acc = acc + diff * diff
            d2_v[pl.ds(off, L)] = acc

        pltpu.sync_copy(d2_v, d2_hbm.at[pl.ds(base, E_PER_W)])
        emb_cp.wait()
        pltpu.sync_copy(rows_v, emb_hbm.at[pl.ds(zbase, Z_PER_W)])

    return sc_fused


_RBF_ROWS = 12800
_N_BLOCKS = N_EDGES // _RBF_ROWS
_PRE_ROWS = N_EDGES // N_FEATURES


def _mu_row():
    mu_i = lax.broadcasted_iota(jnp.int32, (1, N_FEATURES), 1)
    return mu_i.astype(jnp.float32) * jnp.float32(R_CUT / (N_FEATURES - 1))


def _split3(x):
    hi = x.astype(jnp.bfloat16)
    r = x - hi.astype(jnp.float32)
    mid = r.astype(jnp.bfloat16)
    lo = (r - mid.astype(jnp.float32)).astype(jnp.bfloat16)
    return hi, mid, lo


def _pre_body(d2_ref, a_ref):
    d2 = d2_ref[...]
    d = jnp.sqrt(d2 + 1e-12)
    cut = 0.5 * (jnp.cos(jnp.pi * d / R_CUT) + 1.0)
    cut = jnp.where(d < R_CUT, cut, 0.0)
    cut = jnp.maximum(cut, 1e-37)
    b = (2.0 * GAMMA) * d
    h = jnp.log(cut) - GAMMA * (d * d)
    bh, bm, bl = _split3(b)
    hh, hm, hl = _split3(h)
    P = _PRE_ROWS
    a_ref[0 * P:1 * P, :] = bh
    a_ref[1 * P:2 * P, :] = bh
    a_ref[2 * P:3 * P, :] = bm
    a_ref[3 * P:4 * P, :] = bm
    a_ref[4 * P:5 * P, :] = bl
    a_ref[5 * P:6 * P, :] = hh
    a_ref[6 * P:7 * P, :] = hm
    a_ref[7 * P:8 * P, :] = hl


_pre = pl.pallas_call(
    _pre_body,
    out_shape=jax.ShapeDtypeStruct((8 * _PRE_ROWS, N_FEATURES), jnp.bfloat16),
)


def _rbf_body(a_ref, out_ref):
    A = a_ref[...]
    mu = _mu_row()
    mh = mu.astype(jnp.bfloat16)
    ml = (mu - mh.astype(jnp.float32)).astype(jnp.bfloat16)
    ones = jnp.ones((1, N_FEATURES), jnp.bfloat16)
    B = jnp.concatenate([mh, ml, mh, ml, mh, ones, ones, ones], axis=0)
    acc = lax.dot_general(
        A, B, (((0,), (0,)), ((), ())),
        preferred_element_type=jnp.float32,
    )
    g = GAMMA * (mu * mu)
    out_ref[...] = jnp.exp(acc - g)


_rbf = pl.pallas_call(
    _rbf_body,
    grid=(_N_BLOCKS,),
    in_specs=[pl.BlockSpec((8, _RBF_ROWS), lambda i: (0, i))],
    out_specs=pl.BlockSpec((_RBF_ROWS, N_FEATURES), lambda i: (i, 0)),
    out_shape=jax.ShapeDtypeStruct((N_EDGES, N_FEATURES), jnp.float32),
)


def kernel(z, pos, edge_index, batch, emb_table):
    del batch
    z = z.astype(jnp.int32)
    edge_index = edge_index.astype(jnp.int32)
    pos = pos.astype(jnp.float32)
    emb_table = emb_table.astype(jnp.float32)

    z_pad = jnp.concatenate([z, jnp.zeros((Z_PAD - N_NODES,), jnp.int32)])
    d2, emb = _sc_kernel()(pos.reshape(N_NODES * 3),
                           edge_index.reshape(2 * N_EDGES), z_pad, emb_table)
    A = _pre(d2.reshape(_PRE_ROWS, N_FEATURES)).reshape(8, N_EDGES)
    dist_edge = _rbf(A)
    atom_node = emb[:N_NODES]

    force_node = jnp.zeros((N_NODES, 3, N_FEATURES), jnp.float32)
    disp_node = jnp.zeros((N_NODES, 3, N_FEATURES), jnp.float32)
    return (atom_node, force_node, disp_node, dist_edge)


# speedup vs baseline: 12.0075x; 1.0223x over previous
import functools

import jax
import jax.numpy as jnp
from jax import lax
from jax.experimental import pallas as pl
from jax.experimental.pallas import tpu as pltpu
from jax.experimental.pallas import tpu_sc as plsc

N_FEATURES = 128
Z_MAX = 100
R_CUT = 5.0
GAMMA = 10.0
N_NODES = 10000
N_EDGES = 320000

NC, NS, L = 2, 16, 16
NW = NC * NS

E_PER_W = N_EDGES // NW
Z_PAD = 10240
Z_PER_W = Z_PAD // NW


@functools.lru_cache(maxsize=1)
def _sc_kernel():
    mesh = plsc.VectorSubcoreMesh(
        core_axis_name="c", subcore_axis_name="s", num_cores=NC, num_subcores=NS
    )

    @functools.partial(
        pl.kernel,
        out_type=(
            jax.ShapeDtypeStruct((N_EDGES,), jnp.float32),
            jax.ShapeDtypeStruct((N_NODES, N_FEATURES), jnp.float32),
        ),
        mesh=mesh,
        scratch_types=[
            pltpu.VMEM((N_NODES * 3,), jnp.float32),
            pltpu.VMEM((E_PER_W,), jnp.int32),
            pltpu.VMEM((E_PER_W,), jnp.int32),
            pltpu.VMEM((E_PER_W,), jnp.float32),
            pltpu.VMEM((Z_PER_W,), jnp.int32),
            pltpu.VMEM((Z_PER_W, N_FEATURES), jnp.float32),
            pltpu.SemaphoreType.DMA,
        ],
        compiler_params=pltpu.CompilerParams(needs_layout_passes=False),
    )
    def sc_fused(pos_hbm, eidx_hbm, z_hbm, table_hbm, d2_hbm, emb_hbm,
                 pos_v, src_v, dst_v, d2_v, idx_v, rows_v, sem):
        wid = lax.axis_index("s") * NC + lax.axis_index("c")
        zbase = wid * Z_PER_W
        pltpu.sync_copy(z_hbm.at[pl.ds(zbase, Z_PER_W)], idx_v)
        emb_cp = pltpu.async_copy(table_hbm.at[idx_v], rows_v, sem)

        base = wid * E_PER_W
        pltpu.sync_copy(pos_hbm, pos_v)
        pltpu.sync_copy(eidx_hbm.at[pl.ds(base, E_PER_W)], src_v)
        pltpu.sync_copy(eidx_hbm.at[pl.ds(N_EDGES + base, E_PER_W)], dst_v)

        three = jnp.full((L,), 3, jnp.int32)

        @plsc.parallel_loop(0, E_PER_W // L, 1, unroll=8)
        def body(i):
            off = i * L
            s = src_v[pl.ds(off, L)] * three
            t = dst_v[pl.ds(off, L)] * three
            acc = jnp.zeros((L,), jnp.float32)
            for c in range(3):
                col = jnp.full((L,), c, jnp.int32)
                a = plsc.load_gather(pos_v, [s + col])
                b = plsc.load_gather(pos_v, [t + col])
                diff = a - b
                acc = acc + diff * diff
            d2_v[pl.ds(off, L)] = acc

        pltpu.sync_copy(d2_v, d2_hbm.at[pl.ds(base, E_PER_W)])
        emb_cp.wait()

        @pl.when(wid < NW - 1)
        def _():
            pltpu.sync_copy(rows_v, emb_hbm.at[pl.ds(zbase, Z_PER_W)])

        @pl.when(wid == NW - 1)
        def _():
            pltpu.sync_copy(rows_v.at[0:N_NODES - (NW - 1) * Z_PER_W],
                            emb_hbm.at[pl.ds(zbase, N_NODES - (NW - 1) * Z_PER_W)])

    ZW = (N_NODES * 3 * N_FEATURES) // NW
    ZBUF = 15000

    @functools.partial(
        pl.kernel,
        out_type=(
            jax.ShapeDtypeStruct((N_NODES * 3 * N_FEATURES,), jnp.float32),
            jax.ShapeDtypeStruct((N_NODES * 3 * N_FEATURES,), jnp.float32),
        ),
        mesh=mesh,
        scratch_types=[pltpu.VMEM((ZBUF,), jnp.float32)],
        compiler_params=pltpu.CompilerParams(needs_layout_passes=False),
    )
    def sc_zeros(d2_hbm, f_hbm, g_hbm, buf_v):
        wid = lax.axis_index("s") * NC + lax.axis_index("c")
        base = wid * ZW

        @plsc.parallel_loop(0, ZBUF // L, 1, unroll=8)
        def zero(i):
            buf_v[pl.ds(i * L, L)] = jnp.zeros((L,), jnp.float32)

        for k in range(ZW // ZBUF):
            pltpu.sync_copy(buf_v, f_hbm.at[pl.ds(base + k * ZBUF, ZBUF)])
            pltpu.sync_copy(buf_v, g_hbm.at[pl.ds(base + k * ZBUF, ZBUF)])

    return sc_fused, sc_zeros


_RBF_ROWS = 12800
_N_BLOCKS = N_EDGES // _RBF_ROWS
_PRE_ROWS = N_EDGES // N_FEATURES


def _mu_row():
    mu_i = lax.broadcasted_iota(jnp.int32, (1, N_FEATURES), 1)
    return mu_i.astype(jnp.float32) * jnp.float32(R_CUT / (N_FEATURES - 1))


def _split3(x):
    hi = x.astype(jnp.bfloat16)
    r = x - hi.astype(jnp.float32)
    mid = r.astype(jnp.bfloat16)
    lo = (r - mid.astype(jnp.float32)).astype(jnp.bfloat16)
    return hi, mid, lo


def _pre_body(d2_ref, a_ref):
    d2 = d2_ref[...]
    d = jnp.sqrt(d2 + 1e-12)
    cut = 0.5 * (jnp.cos(jnp.pi * d / R_CUT) + 1.0)
    cut = jnp.where(d < R_CUT, cut, 0.0)
    cut = jnp.maximum(cut, 1e-37)
    b = (2.0 * GAMMA) * d
    h = jnp.log(cut) - GAMMA * (d * d)
    bh, bm, bl = _split3(b)
    hh, hm, hl = _split3(h)
    P = _PRE_ROWS
    a_ref[0 * P:1 * P, :] = bh
    a_ref[1 * P:2 * P, :] = bh
    a_ref[2 * P:3 * P, :] = bm
    a_ref[3 * P:4 * P, :] = bm
    a_ref[4 * P:5 * P, :] = bl
    a_ref[5 * P:6 * P, :] = hh
    a_ref[6 * P:7 * P, :] = hm
    a_ref[7 * P:8 * P, :] = hl


_pre = pl.pallas_call(
    _pre_body,
    out_shape=jax.ShapeDtypeStruct((8 * _PRE_ROWS, N_FEATURES), jnp.bfloat16),
)


def _rbf_body(a_ref, out_ref):
    A = a_ref[...]
    mu = _mu_row()
    mh = mu.astype(jnp.bfloat16)
    ml = (mu - mh.astype(jnp.float32)).astype(jnp.bfloat16)
    ones = jnp.ones((1, N_FEATURES), jnp.bfloat16)
    B = jnp.concatenate([mh, ml, mh, ml, mh, ones, ones, ones], axis=0)
    acc = lax.dot_general(
        A, B, (((0,), (0,)), ((), ())),
        preferred_element_type=jnp.float32,
    )
    g = GAMMA * (mu * mu)
    out_ref[...] = jnp.exp(acc - g)


_rbf = pl.pallas_call(
    _rbf_body,
    grid=(_N_BLOCKS,),
    in_specs=[pl.BlockSpec((8, _RBF_ROWS), lambda i: (0, i))],
    out_specs=pl.BlockSpec((_RBF_ROWS, N_FEATURES), lambda i: (i, 0)),
    out_shape=jax.ShapeDtypeStruct((N_EDGES, N_FEATURES), jnp.float32),
)


def kernel(z, pos, edge_index, batch, emb_table):
    del batch
    z = z.astype(jnp.int32)
    edge_index = edge_index.astype(jnp.int32)
    pos = pos.astype(jnp.float32)
    emb_table = emb_table.astype(jnp.float32)

    z_pad = jnp.concatenate([z, jnp.zeros((Z_PAD - N_NODES,), jnp.int32)])
    sc_fused, sc_zeros = _sc_kernel()
    d2, atom_node = sc_fused(pos.reshape(N_NODES * 3),
                             edge_index.reshape(2 * N_EDGES), z_pad, emb_table)
    A = _pre(d2.reshape(_PRE_ROWS, N_FEATURES)).reshape(8, N_EDGES)
    dist_edge = _rbf(A)

    del sc_zeros
    force_node = jnp.zeros((N_NODES, 3, N_FEATURES), jnp.float32)
    disp_node = jnp.zeros((N_NODES, 3, N_FEATURES), jnp.float32)
    return (atom_node, force_node, disp_node, dist_edge)


# speedup vs baseline: 12.0412x; 1.0028x over previous
import functools

import jax
import jax.numpy as jnp
from jax import lax
from jax.experimental import pallas as pl
from jax.experimental.pallas import tpu as pltpu
from jax.experimental.pallas import tpu_sc as plsc

N_FEATURES = 128
Z_MAX = 100
R_CUT = 5.0
GAMMA = 10.0
N_NODES = 10000
N_EDGES = 320000

NC, NS, L = 2, 16, 16
NW = NC * NS

E_PER_W = N_EDGES // NW
Z_PAD = 10240
Z_PER_W = Z_PAD // NW


@functools.lru_cache(maxsize=1)
def _sc_kernel():
    mesh = plsc.VectorSubcoreMesh(
        core_axis_name="c", subcore_axis_name="s", num_cores=NC, num_subcores=NS
    )

    @functools.partial(
        pl.kernel,
        out_type=(
            jax.ShapeDtypeStruct((N_EDGES,), jnp.float32),
            jax.ShapeDtypeStruct((N_NODES, N_FEATURES), jnp.float32),
        ),
        mesh=mesh,
        scratch_types=[
            pltpu.VMEM((N_NODES * 3,), jnp.float32),
            pltpu.VMEM((E_PER_W,), jnp.int32),
            pltpu.VMEM((E_PER_W,), jnp.int32),
            pltpu.VMEM((E_PER_W,), jnp.float32),
            pltpu.VMEM((Z_PER_W,), jnp.int32),
            pltpu.VMEM((Z_PER_W, N_FEATURES), jnp.float32),
            pltpu.SemaphoreType.DMA,
        ],
        compiler_params=pltpu.CompilerParams(needs_layout_passes=False),
    )
    def sc_fused(pos_hbm, eidx_hbm, z_hbm, table_hbm, d2_hbm, emb_hbm,
                 pos_v, src_v, dst_v, d2_v, idx_v, rows_v, sem):
        wid = lax.axis_index("s") * NC + lax.axis_index("c")
        zbase = wid * Z_PER_W
        pltpu.sync_copy(z_hbm.at[pl.ds(zbase, Z_PER_W)], idx_v)
        emb_cp = pltpu.async_copy(table_hbm.at[idx_v], rows_v, sem)

        base = wid * E_PER_W
        pltpu.sync_copy(pos_hbm, pos_v)
        pltpu.sync_copy(eidx_hbm.at[pl.ds(base, E_PER_W)], src_v)
        pltpu.sync_copy(eidx_hbm.at[pl.ds(N_EDGES + base, E_PER_W)], dst_v)

        three = jnp.full((L,), 3, jnp.int32)

        @plsc.parallel_loop(0, E_PER_W // L, 1, unroll=8)
        def body(i):
            off = i * L
            s = src_v[pl.ds(off, L)] * three
            t = dst_v[pl.ds(off, L)] * three
            acc = jnp.zeros((L,), jnp.float32)
            for c in range(3):
                col = jnp.full((L,), c, jnp.int32)
                a = plsc.load_gather(pos_v, [s + col])
                b = plsc.load_gather(pos_v, [t + col])
                diff = a - b
                acc = acc + diff * diff
            d2_v[pl.ds(off, L)] = acc

        pltpu.sync_copy(d2_v, d2_hbm.at[pl.ds(base, E_PER_W)])
        emb_cp.wait()

        @pl.when(wid < NW - 1)
        def _():
            pltpu.sync_copy(rows_v, emb_hbm.at[pl.ds(zbase, Z_PER_W)])

        @pl.when(wid == NW - 1)
        def _():
            pltpu.sync_copy(rows_v.at[0:N_NODES - (NW - 1) * Z_PER_W],
                            emb_hbm.at[pl.ds(zbase, N_NODES - (NW - 1) * Z_PER_W)])

    return sc_fused


_RBF_ROWS = 12800
_N_BLOCKS = N_EDGES // _RBF_ROWS
_PRE_ROWS = N_EDGES // N_FEATURES


def _mu_row():
    mu_i = lax.broadcasted_iota(jnp.int32, (1, N_FEATURES), 1)
    return mu_i.astype(jnp.float32) * jnp.float32(R_CUT / (N_FEATURES - 1))


def _split3(x):
    hi = x.astype(jnp.bfloat16)
    r = x - hi.astype(jnp.float32)
    mid = r.astype(jnp.bfloat16)
    lo = (r - mid.astype(jnp.float32)).astype(jnp.bfloat16)
    return hi, mid, lo


def _pre_body(d2_ref, a_ref):
    d2 = d2_ref[...]
    d = jnp.sqrt(d2 + 1e-12)
    cut = 0.5 * (jnp.cos(jnp.pi * d / R_CUT) + 1.0)
    cut = jnp.where(d < R_CUT, cut, 0.0)
    cut = jnp.maximum(cut, 1e-37)
    b = (2.0 * GAMMA) * d
    h = jnp.log(cut) - GAMMA * (d * d)
    bh, bm, bl = _split3(b)
    hh, hm, hl = _split3(h)
    P = _PRE_ROWS
    a_ref[0 * P:1 * P, :] = bh
    a_ref[1 * P:2 * P, :] = bh
    a_ref[2 * P:3 * P, :] = bm
    a_ref[3 * P:4 * P, :] = bm
    a_ref[4 * P:5 * P, :] = bl
    a_ref[5 * P:6 * P, :] = hh
    a_ref[6 * P:7 * P, :] = hm
    a_ref[7 * P:8 * P, :] = hl


_pre = pl.pallas_call(
    _pre_body,
    out_shape=jax.ShapeDtypeStruct((8 * _PRE_ROWS, N_FEATURES), jnp.bfloat16),
)


def _rbf_body(a_ref, out_ref):
    A = a_ref[...]
    mu = _mu_row()
    mh = mu.astype(jnp.bfloat16)
    ml = (mu - mh.astype(jnp.float32)).astype(jnp.bfloat16)
    ones = jnp.ones((1, N_FEATURES), jnp.bfloat16)
    B = jnp.concatenate([mh, ml, mh, ml, mh, ones, ones, ones], axis=0)
    acc = lax.dot_general(
        A, B, (((0,), (0,)), ((), ())),
        preferred_element_type=jnp.float32,
    )
    g = GAMMA * (mu * mu)
    out_ref[...] = jnp.exp(acc - g)


_rbf = pl.pallas_call(
    _rbf_body,
    grid=(_N_BLOCKS,),
    in_specs=[pl.BlockSpec((8, _RBF_ROWS), lambda i: (0, i))],
    out_specs=pl.BlockSpec((_RBF_ROWS, N_FEATURES), lambda i: (i, 0)),
    out_shape=jax.ShapeDtypeStruct((N_EDGES, N_FEATURES), jnp.float32),
)


def kernel(z, pos, edge_index, batch, emb_table):
    del batch
    z = z.astype(jnp.int32)
    edge_index = edge_index.astype(jnp.int32)
    pos = pos.astype(jnp.float32)
    emb_table = emb_table.astype(jnp.float32)

    z_pad = jnp.concatenate([z, jnp.zeros((Z_PAD - N_NODES,), jnp.int32)])
    d2, atom_node = _sc_kernel()(pos.reshape(N_NODES * 3),
                                 edge_index.reshape(2 * N_EDGES), z_pad,
                                 emb_table)
    A = _pre(d2.reshape(_PRE_ROWS, N_FEATURES)).reshape(8, N_EDGES)
    dist_edge = _rbf(A)

    force_node = jnp.zeros((N_NODES, 3, N_FEATURES), jnp.float32)
    disp_node = jnp.zeros((N_NODES, 3, N_FEATURES), jnp.float32)
    return (atom_node, force_node, disp_node, dist_edge)
